# baseline (speedup 1.0000x reference)
import functools
import numpy as np
import jax
import jax.numpy as jnp
from jax import lax
from jax.experimental import pallas as pl
from jax.experimental.pallas import tpu as pltpu
from jax.experimental.pallas import tpu_sc as plsc

B = 128
L = 4096
PATCH = 32
P = L // PATCH
G = B * P
E = 440
GPP = 16

PERIOD = E * GPP
NPERIODS = (G * E) // PERIOD
NTILES = 32
PPT = NPERIODS // NTILES
NBV = G * PATCH
BVPT = NBV // NTILES

EI_TILES_PER_ROW = NTILES // 2
EI_SLOTS_PER_TILE = NPERIODS // EI_TILES_PER_ROW
EI_STAGE_SLOTS = 2
EI_FIRES = EI_SLOTS_PER_TILE // EI_STAGE_SLOTS

_sc_mesh = plsc.VectorSubcoreMesh(core_axis_name="c", subcore_axis_name="s")


@functools.partial(
    pl.kernel,
    out_type=(
        jax.ShapeDtypeStruct((2, G * E), jnp.int32),
        jax.ShapeDtypeStruct((G * E,), jnp.float32),
        jax.ShapeDtypeStruct((NBV,), jnp.int32),
    ),
    mesh=_sc_mesh,
    compiler_params=pltpu.CompilerParams(needs_layout_passes=False),
    scratch_types=(
        pltpu.VMEM((8,), jnp.float32),
        pltpu.VMEM((E,), jnp.int32),
        pltpu.VMEM((E,), jnp.int32),
        pltpu.VMEM((PERIOD,), jnp.float32),
        pltpu.VMEM((PERIOD,), jnp.int32),
        pltpu.VMEM((EI_STAGE_SLOTS * PERIOD,), jnp.int32),
        pltpu.VMEM((EI_STAGE_SLOTS * PERIOD,), jnp.int32),
        pltpu.VMEM((BVPT,), jnp.int32),
        pltpu.SemaphoreType.DMA,
        pltpu.SemaphoreType.DMA,
    ),
)
def _sc_build(w_hbm, dist_hbm, base_hbm, ei_hbm, ea_hbm, bv_hbm,
              w_v, dist_v, row_v, attr_v, eip_v, stag_a, stag_b, bv_v,
              ea_sem, ei_sem):
    c = lax.axis_index("c")
    s = lax.axis_index("s")
    t = s * 2 + c
    row = t // EI_TILES_PER_ROW
    slot0 = (t % EI_TILES_PER_ROW) * EI_SLOTS_PER_TILE
    pltpu.sync_copy(w_hbm, w_v)
    pltpu.sync_copy(dist_hbm, dist_v)
    pltpu.sync_copy(base_hbm.at[row], row_v)
    iota = lax.iota(jnp.int32, 16)
    e_div = jnp.full((16,), E, jnp.int32)

    def build_periods(i, carry):
        k = iota + i * 16
        e = lax.rem(k, e_div)
        g = lax.div(k, e_div)
        d = plsc.load_gather(dist_v, [e])
        a = plsc.load_gather(w_v, [d])
        plsc.store_scatter(attr_v, [k], a)
        bg = plsc.load_gather(row_v, [e])
        plsc.store_scatter(eip_v, [k], bg + g * PATCH)
        return carry

    lax.fori_loop(0, PERIOD // 16, build_periods, 0)

    def build_bv(i, carry):
        j = iota + i * 16
        k = j + t * BVPT
        plsc.store_scatter(bv_v, [j], lax.shift_right_logical(k, 5))
        return carry

    lax.fori_loop(0, BVPT // 16, build_bv, 0)

    def fire_ea(j, carry):
        pltpu.async_copy(attr_v,
                         ea_hbm.at[pl.ds((t * PPT + j) * PERIOD, PERIOD)],
                         ea_sem)
        return carry

    lax.fori_loop(0, PPT, fire_ea, 0)
    pltpu.sync_copy(bv_v, bv_hbm.at[pl.ds(t * BVPT, BVPT)])

    def _bake_and_fire(stag, sbase):
        for p in range(EI_STAGE_SLOTS):
            delta = (sbase + p) * (GPP * PATCH)
            for i in range(PERIOD // 16):
                src = eip_v[pl.ds(i * 16, 16)]
                stag[pl.ds((p * (PERIOD // 16) + i) * 16, 16)] = src + delta
        pltpu.async_copy(
            stag,
            ei_hbm.at[row, pl.ds(sbase * PERIOD, EI_STAGE_SLOTS * PERIOD)],
            ei_sem)

    def _wait_ei():
        pltpu.make_async_copy(
            stag_a,
            ei_hbm.at[row, pl.ds(slot0 * PERIOD, EI_STAGE_SLOTS * PERIOD)],
            ei_sem).wait()

    def fire_ei(j, carry):
        sbase = slot0 + j * EI_STAGE_SLOTS

        @pl.when(j >= 2)
        def _():
            _wait_ei()

        @pl.when(j % 2 == 0)
        def _():
            _bake_and_fire(stag_a, sbase)

        @pl.when(j % 2 == 1)
        def _():
            _bake_and_fire(stag_b, sbase)

        return carry

    lax.fori_loop(0, EI_FIRES, fire_ei, 0)
    _wait_ei()
    _wait_ei()

    def drain_ea(j, carry):
        pltpu.make_async_copy(
            attr_v, ea_hbm.at[pl.ds(t * PPT * PERIOD, PERIOD)],
            ea_sem).wait()
        return carry

    lax.fori_loop(0, PPT, drain_ea, 0)


def kernel(iq_signal, edge_weights, edge_index_base, edge_distance):
    edge_index, edge_attr, batch_vec = _sc_build(
        edge_weights, edge_distance, edge_index_base)
    node_features = jnp.transpose(iq_signal, (0, 2, 1)).reshape(B * L, 2)
    return node_features, edge_index, edge_attr, batch_vec


# speedup vs baseline: 1.0011x; 1.0011x over previous
import functools
import jax
import jax.numpy as jnp
from jax import lax
from jax.experimental import pallas as pl
from jax.experimental.pallas import tpu as pltpu
from jax.experimental.pallas import tpu_sc as plsc

B = 128
L = 4096
PATCH = 32
P = L // PATCH
G = B * P
E = 440
GPP = 16

PERIOD = E * GPP
NPERIODS = (G * E) // PERIOD
NTILES = 32
PPT = NPERIODS // NTILES
NBV = G * PATCH
BVPT = NBV // NTILES

EI_TILES_PER_ROW = NTILES // 2
EI_SLOTS_PER_TILE = NPERIODS // EI_TILES_PER_ROW
EI_STAGE_SLOTS = 2
EI_FIRES = EI_SLOTS_PER_TILE // EI_STAGE_SLOTS

_sc_mesh = plsc.VectorSubcoreMesh(core_axis_name="c", subcore_axis_name="s")


@functools.partial(
    pl.kernel,
    out_type=(
        jax.ShapeDtypeStruct((2, G * E), jnp.int32),
        jax.ShapeDtypeStruct((G * E,), jnp.float32),
        jax.ShapeDtypeStruct((NBV,), jnp.int32),
    ),
    mesh=_sc_mesh,
    compiler_params=pltpu.CompilerParams(needs_layout_passes=False),
    scratch_types=(
        pltpu.VMEM((8,), jnp.float32),
        pltpu.VMEM((E,), jnp.int32),
        pltpu.VMEM((E,), jnp.int32),
        pltpu.VMEM((PERIOD,), jnp.float32),
        pltpu.VMEM((PERIOD,), jnp.int32),
        pltpu.VMEM((EI_STAGE_SLOTS * PERIOD,), jnp.int32),
        pltpu.VMEM((EI_STAGE_SLOTS * PERIOD,), jnp.int32),
        pltpu.VMEM((BVPT,), jnp.int32),
        pltpu.SemaphoreType.DMA,
        pltpu.SemaphoreType.DMA,
    ),
)
def _sc_build(w_hbm, dist_hbm, base_hbm, ei_hbm, ea_hbm, bv_hbm,
              w_v, dist_v, row_v, attr_v, eip_v, stag_a, stag_b, bv_v,
              ea_sem, ei_sem):
    c = lax.axis_index("c")
    s = lax.axis_index("s")
    t = s * 2 + c
    row = t // EI_TILES_PER_ROW
    slot0 = (t % EI_TILES_PER_ROW) * EI_SLOTS_PER_TILE
    pltpu.sync_copy(w_hbm, w_v)
    pltpu.sync_copy(dist_hbm, dist_v)
    pltpu.sync_copy(base_hbm.at[row], row_v)
    iota = lax.iota(jnp.int32, 16)
    e_div = jnp.full((16,), E, jnp.int32)

    def build_periods(i, carry):
        k = iota + i * 16
        e = lax.rem(k, e_div)
        g = lax.div(k, e_div)
        d = plsc.load_gather(dist_v, [e])
        a = plsc.load_gather(w_v, [d])
        plsc.store_scatter(attr_v, [k], a)
        bg = plsc.load_gather(row_v, [e])
        plsc.store_scatter(eip_v, [k], bg + g * PATCH)
        return carry

    lax.fori_loop(0, PERIOD // 16, build_periods, 0)

    def build_bv(i, carry):
        j = iota + i * 16
        k = j + t * BVPT
        plsc.store_scatter(bv_v, [j], lax.shift_right_logical(k, 5))
        return carry

    lax.fori_loop(0, BVPT // 16, build_bv, 0)

    def fire_ea(j, carry):
        pltpu.async_copy(attr_v,
                         ea_hbm.at[pl.ds((t * PPT + j) * PERIOD, PERIOD)],
                         ea_sem)
        return carry

    lax.fori_loop(0, PPT, fire_ea, 0)
    pltpu.sync_copy(bv_v, bv_hbm.at[pl.ds(t * BVPT, BVPT)])

    def _bake_and_fire(stag, sbase):
        for p in range(EI_STAGE_SLOTS):
            delta = (sbase + p) * (GPP * PATCH)
            for i in range(PERIOD // 16):
                src = eip_v[pl.ds(i * 16, 16)]
                stag[pl.ds((p * (PERIOD // 16) + i) * 16, 16)] = src + delta
        pltpu.async_copy(
            stag,
            ei_hbm.at[row, pl.ds(sbase * PERIOD, EI_STAGE_SLOTS * PERIOD)],
            ei_sem)

    def _wait_ei():
        pltpu.make_async_copy(
            stag_a,
            ei_hbm.at[row, pl.ds(slot0 * PERIOD, EI_STAGE_SLOTS * PERIOD)],
            ei_sem).wait()

    def fire_ei(j, carry):
        sbase = slot0 + j * EI_STAGE_SLOTS

        @pl.when(j >= 2)
        def _():
            _wait_ei()

        @pl.when(j % 2 == 0)
        def _():
            _bake_and_fire(stag_a, sbase)

        @pl.when(j % 2 == 1)
        def _():
            _bake_and_fire(stag_b, sbase)

        return carry

    lax.fori_loop(0, EI_FIRES, fire_ei, 0)
    _wait_ei()
    _wait_ei()

    def drain_ea(j, carry):
        pltpu.make_async_copy(
            attr_v, ea_hbm.at[pl.ds(t * PPT * PERIOD, PERIOD)],
            ea_sem).wait()
        return carry

    lax.fori_loop(0, PPT, drain_ea, 0)


def kernel(iq_signal, edge_weights, edge_index_base, edge_distance):
    edge_index, edge_attr, batch_vec = _sc_build(
        edge_weights, edge_distance, edge_index_base)
    node_features = jnp.transpose(iq_signal, (0, 2, 1)).reshape(B * L, 2)
    return node_features, edge_index, edge_attr, batch_vec
